# baseline (speedup 1.0000x reference)
import jax
import jax.numpy as jnp
from jax import lax
from jax.experimental import pallas as pl
from jax.experimental.pallas import tpu as pltpu
from jax.experimental.pallas import tpu_sc as plsc

C = 1000
NB = 3
B = 1024
M = 50
RATES = (1.0, 0.5, 0.1)

NCORES = 2
NSUB = 16
LANES = 16
NW = NCORES * NSUB
BPT = B // NW
CPAD = 1024
SROW = NB * CPAD
NIDX = BPT * M
GCHUNK = 128
NGC = 13
NIDX_PAD = NGC * GCHUNK
NGROUP = BPT // LANES

_SC_PARAMS = pltpu.CompilerParams(needs_layout_passes=False)
_SC_MESH = plsc.VectorSubcoreMesh(core_axis_name="c", subcore_axis_name="s")


def _scatter_body(cjs_hbm, ti_hbm, tjs_hbm, s_out,
                  cj_v, tj_v, ti_v, s_v, sem):
    wid = lax.axis_index("s") * NCORES + lax.axis_index("c")
    iota = lax.broadcasted_iota(jnp.int32, (LANES,), 0)
    zeros = jnp.zeros((LANES,), jnp.int32)

    cp = [pltpu.async_copy(cjs_hbm.at[pl.ds(wid * BPT, BPT)], cj_v, sem),
          pltpu.async_copy(tjs_hbm.at[pl.ds(wid * BPT, BPT)], tj_v, sem),
          pltpu.async_copy(ti_hbm.at[pl.ds(wid * BPT, BPT)], ti_v, sem)]

    for b in range(BPT):
        def zero(i, _, b=b):
            plsc.store_scatter(s_v, [jnp.full((LANES,), b, jnp.int32),
                                     i * 16 + iota],
                               jnp.zeros((LANES,), jnp.float32))
            return 0

        lax.fori_loop(0, SROW // LANES, zero, 0, unroll=8)
    for c in cp:
        c.wait()

    for g in range(NGROUP):
        lane_row = g * LANES + iota
        ti_g = plsc.load_gather(ti_v, [lane_row, zeros])
        tlast = plsc.load_gather(tj_v, [lane_row, zeros + (M - 1)])

        def scat(j, _, ti_g=ti_g, tlast=tlast, lane_row=lane_row):
            cj = plsc.load_gather(cj_v, [lane_row, zeros + j])
            tj = plsc.load_gather(tj_v, [lane_row, zeros + j])
            dt = ti_g - tj
            ts = tlast - tj
            for m in range(NB):
                r = RATES[m]
                plsc.addupdate_scatter(s_v, [lane_row, m * CPAD + cj],
                                       jnp.exp(-r * ts) - jnp.exp(-r * dt))
            return 0

        lax.fori_loop(0, M, scat, 0)

    pltpu.sync_copy(s_v, s_out.at[pl.ds(wid * BPT, BPT)])


_scatter_call = pl.kernel(
    _scatter_body,
    out_type=jax.ShapeDtypeStruct((B, SROW), jnp.float32),
    mesh=_SC_MESH,
    compiler_params=_SC_PARAMS,
    scratch_types=[
        pltpu.VMEM((BPT, M), jnp.int32),
        pltpu.VMEM((BPT, M), jnp.float32),
        pltpu.VMEM((BPT, 1), jnp.float32),
        pltpu.VMEM((BPT, SROW), jnp.float32),
        pltpu.SemaphoreType.DMA,
    ],
)


def _phi_body(ci_hbm, cjs_hbm, ti_hbm, tjs_hbm, wall_hbm,
              phi_out,
              ci_v, cif_v, cj_v, tj_v, ti_v, wrows_v, phi_v, sem):
    wid = lax.axis_index("s") * NCORES + lax.axis_index("c")
    iota = lax.broadcasted_iota(jnp.int32, (LANES,), 0)
    zeros = jnp.zeros((LANES,), jnp.int32)

    pltpu.sync_copy(ci_hbm.at[pl.ds(wid * BPT, BPT)], ci_v)
    for g in range(NGROUP):
        lane_row = g * LANES + iota
        ci_g = plsc.load_gather(ci_v, [lane_row, zeros])
        for m in range(NB):
            plsc.store_scatter(cif_v, [m * BPT + lane_row], ci_g + m * C)

    cp = pltpu.async_copy(wall_hbm.at[cif_v], wrows_v, sem)

    pltpu.sync_copy(cjs_hbm.at[pl.ds(wid * BPT, BPT)], cj_v)
    pltpu.sync_copy(tjs_hbm.at[pl.ds(wid * BPT, BPT)], tj_v)
    pltpu.sync_copy(ti_hbm.at[pl.ds(wid * BPT, BPT)], ti_v)
    cp.wait()

    for g in range(NGROUP):
        lane_row = g * LANES + iota
        ti_g = plsc.load_gather(ti_v, [lane_row, zeros])

        def dot(j, acc, ti_g=ti_g, lane_row=lane_row):
            tj = plsc.load_gather(tj_v, [lane_row, zeros + j])
            cj = plsc.load_gather(cj_v, [lane_row, zeros + j])
            dt = ti_g - tj
            for m in range(NB):
                r = RATES[m]
                acc = acc + (plsc.load_gather(wrows_v, [m * BPT + lane_row, cj])
                             * (r * jnp.exp(-r * dt)))
            return acc

        acc = lax.fori_loop(0, M, dot, jnp.zeros((LANES,), jnp.float32))
        phi_v[pl.ds(g * LANES, LANES)] = acc
    pltpu.sync_copy(phi_v, phi_out.at[pl.ds(wid * BPT, BPT)])


_phi_call = pl.kernel(
    _phi_body,
    out_type=jax.ShapeDtypeStruct((B,), jnp.float32),
    mesh=_SC_MESH,
    compiler_params=pltpu.CompilerParams(needs_layout_passes=False,
                                         use_tc_tiling_on_sc=False),
    scratch_types=[
        pltpu.VMEM((BPT, 1), jnp.int32),
        pltpu.VMEM((NB * BPT,), jnp.int32),
        pltpu.VMEM((BPT, M), jnp.int32),
        pltpu.VMEM((BPT, M), jnp.float32),
        pltpu.VMEM((BPT, 1), jnp.float32),
        pltpu.VMEM((NB * BPT, C), jnp.float32),
        pltpu.VMEM((BPT,), jnp.float32),
        pltpu.SemaphoreType.DMA,
    ],
)


def _mm_body(s_ref, w0_ref, w1_ref, w2_ref, o_ref):
    s = s_ref[:]
    dn = (((1,), (1,)), ((), ()))
    acc = lax.dot_general(s[:, :C], w0_ref[:], dn,
                          preferred_element_type=jnp.float32)
    acc = acc + lax.dot_general(s[:, CPAD:CPAD + C], w1_ref[:], dn,
                                preferred_element_type=jnp.float32)
    acc = acc + lax.dot_general(s[:, 2 * CPAD:2 * CPAD + C], w2_ref[:], dn,
                                preferred_element_type=jnp.float32)
    o_ref[:] = acc


_BM = 256
_mm_call = pl.pallas_call(
    _mm_body,
    grid=(B // _BM,),
    in_specs=[
        pl.BlockSpec((_BM, SROW), lambda i: (i, 0)),
        pl.BlockSpec((C, C), lambda i: (0, 0)),
        pl.BlockSpec((C, C), lambda i: (0, 0)),
        pl.BlockSpec((C, C), lambda i: (0, 0)),
    ],
    out_specs=pl.BlockSpec((_BM, C), lambda i: (i, 0)),
    out_shape=jax.ShapeDtypeStruct((B, C), jnp.float32),
)


def kernel(ci, cjs, ti, tjs, Cs, W0, W1, W2):
    del Cs
    ci = ci.astype(jnp.int32)
    cjs = cjs.astype(jnp.int32)
    s2d = _scatter_call(cjs, ti, tjs)
    wall = jnp.concatenate([W0, W1, W2], axis=0)
    phi = _phi_call(ci, cjs, ti, tjs, wall)
    pHi = _mm_call(s2d, W0, W1, W2)
    return phi.reshape(B, 1), pHi


# speedup vs baseline: 1.1499x; 1.1499x over previous
import jax
import jax.numpy as jnp
from jax import lax
from jax.experimental import pallas as pl
from jax.experimental.pallas import tpu as pltpu
from jax.experimental.pallas import tpu_sc as plsc

C = 1000
NB = 3
B = 1024
M = 50
RATES = (1.0, 0.5, 0.1)

NCORES = 2
NSUB = 16
LANES = 16
NW = NCORES * NSUB
BPT = B // NW
CPAD = 1024
SROW = NB * CPAD
NIDX = BPT * M
GCHUNK = 128
NGC = 13
NIDX_PAD = NGC * GCHUNK
NGROUP = BPT // LANES

_SC_PARAMS = pltpu.CompilerParams(needs_layout_passes=False)
_SC_MESH = plsc.VectorSubcoreMesh(core_axis_name="c", subcore_axis_name="s")


def _scatter_body(cjs_hbm, ti_hbm, tjs_hbm, s_out,
                  cj_v, tj_v, ti_v, s_v, sem):
    wid = lax.axis_index("s") * NCORES + lax.axis_index("c")
    iota = lax.broadcasted_iota(jnp.int32, (LANES,), 0)
    zeros = jnp.zeros((LANES,), jnp.int32)

    cp = [pltpu.async_copy(cjs_hbm.at[pl.ds(wid * BPT, BPT)], cj_v, sem),
          pltpu.async_copy(tjs_hbm.at[pl.ds(wid * BPT, BPT)], tj_v, sem),
          pltpu.async_copy(ti_hbm.at[pl.ds(wid * BPT, BPT)], ti_v, sem)]

    for b in range(BPT):
        def zero(i, _, b=b):
            plsc.store_scatter(s_v, [jnp.full((LANES,), b, jnp.int32),
                                     i * 16 + iota],
                               jnp.zeros((LANES,), jnp.float32))
            return 0

        lax.fori_loop(0, SROW // LANES, zero, 0, unroll=8)
    for c in cp:
        c.wait()

    for g in range(NGROUP):
        lane_row = g * LANES + iota
        ti_g = plsc.load_gather(ti_v, [lane_row, zeros])
        tlast = plsc.load_gather(tj_v, [lane_row, zeros + (M - 1)])

        def scat(j, _, ti_g=ti_g, tlast=tlast, lane_row=lane_row):
            cj = plsc.load_gather(cj_v, [lane_row, zeros + j])
            tj = plsc.load_gather(tj_v, [lane_row, zeros + j])
            dt = ti_g - tj
            ts = tlast - tj
            for m in range(NB):
                r = RATES[m]
                plsc.addupdate_scatter(s_v, [lane_row, m * CPAD + cj],
                                       jnp.exp(-r * ts) - jnp.exp(-r * dt))
            return 0

        lax.fori_loop(0, M, scat, 0)

    pltpu.sync_copy(s_v, s_out.at[pl.ds(wid * BPT, BPT)])


_scatter_call = pl.kernel(
    _scatter_body,
    out_type=jax.ShapeDtypeStruct((B, SROW), jnp.float32),
    mesh=_SC_MESH,
    compiler_params=_SC_PARAMS,
    scratch_types=[
        pltpu.VMEM((BPT, M), jnp.int32),
        pltpu.VMEM((BPT, M), jnp.float32),
        pltpu.VMEM((BPT, 1), jnp.float32),
        pltpu.VMEM((BPT, SROW), jnp.float32),
        pltpu.SemaphoreType.DMA,
    ],
)


def _phi_body(ci_hbm, cjs_hbm, ti_hbm, tjs_hbm, w0_hbm, w1_hbm, w2_hbm,
              phi_out,
              ci_v, cif_v, cj_v, tj_v, ti_v, wrows_v, phi_v, sem):
    wid = lax.axis_index("s") * NCORES + lax.axis_index("c")
    iota = lax.broadcasted_iota(jnp.int32, (LANES,), 0)
    zeros = jnp.zeros((LANES,), jnp.int32)

    pltpu.sync_copy(ci_hbm.at[pl.ds(wid * BPT, BPT)], ci_v)
    for g in range(NGROUP):
        lane_row = g * LANES + iota
        plsc.store_scatter(cif_v, [lane_row],
                           plsc.load_gather(ci_v, [lane_row, zeros]))

    copies = [
        pltpu.async_copy(w_hbm.at[cif_v], wrows_v.at[pl.ds(m * BPT, BPT)], sem)
        for m, w_hbm in enumerate((w0_hbm, w1_hbm, w2_hbm))
    ]

    pltpu.sync_copy(cjs_hbm.at[pl.ds(wid * BPT, BPT)], cj_v)
    pltpu.sync_copy(tjs_hbm.at[pl.ds(wid * BPT, BPT)], tj_v)
    pltpu.sync_copy(ti_hbm.at[pl.ds(wid * BPT, BPT)], ti_v)
    for c in copies:
        c.wait()

    for g in range(NGROUP):
        lane_row = g * LANES + iota
        ti_g = plsc.load_gather(ti_v, [lane_row, zeros])

        def dot(j, acc, ti_g=ti_g, lane_row=lane_row):
            tj = plsc.load_gather(tj_v, [lane_row, zeros + j])
            cj = plsc.load_gather(cj_v, [lane_row, zeros + j])
            dt = ti_g - tj
            for m in range(NB):
                r = RATES[m]
                acc = acc + (plsc.load_gather(wrows_v, [m * BPT + lane_row, cj])
                             * (r * jnp.exp(-r * dt)))
            return acc

        acc = lax.fori_loop(0, M, dot, jnp.zeros((LANES,), jnp.float32))
        phi_v[pl.ds(g * LANES, LANES)] = acc
    pltpu.sync_copy(phi_v, phi_out.at[pl.ds(wid * BPT, BPT)])


_phi_call = pl.kernel(
    _phi_body,
    out_type=jax.ShapeDtypeStruct((B,), jnp.float32),
    mesh=_SC_MESH,
    compiler_params=pltpu.CompilerParams(needs_layout_passes=False,
                                         use_tc_tiling_on_sc=False),
    scratch_types=[
        pltpu.VMEM((BPT, 1), jnp.int32),
        pltpu.VMEM((BPT,), jnp.int32),
        pltpu.VMEM((BPT, M), jnp.int32),
        pltpu.VMEM((BPT, M), jnp.float32),
        pltpu.VMEM((BPT, 1), jnp.float32),
        pltpu.VMEM((NB * BPT, C), jnp.float32),
        pltpu.VMEM((BPT,), jnp.float32),
        pltpu.SemaphoreType.DMA,
    ],
)


def _mm_body(s_ref, w0_ref, w1_ref, w2_ref, o_ref):
    s = s_ref[:]
    dn = (((1,), (1,)), ((), ()))
    acc = lax.dot_general(s[:, :C], w0_ref[:], dn,
                          preferred_element_type=jnp.float32)
    acc = acc + lax.dot_general(s[:, CPAD:CPAD + C], w1_ref[:], dn,
                                preferred_element_type=jnp.float32)
    acc = acc + lax.dot_general(s[:, 2 * CPAD:2 * CPAD + C], w2_ref[:], dn,
                                preferred_element_type=jnp.float32)
    o_ref[:] = acc


_BM = 256
_mm_call = pl.pallas_call(
    _mm_body,
    grid=(B // _BM,),
    in_specs=[
        pl.BlockSpec((_BM, SROW), lambda i: (i, 0)),
        pl.BlockSpec((C, C), lambda i: (0, 0)),
        pl.BlockSpec((C, C), lambda i: (0, 0)),
        pl.BlockSpec((C, C), lambda i: (0, 0)),
    ],
    out_specs=pl.BlockSpec((_BM, C), lambda i: (i, 0)),
    out_shape=jax.ShapeDtypeStruct((B, C), jnp.float32),
)


def kernel(ci, cjs, ti, tjs, Cs, W0, W1, W2):
    del Cs
    ci = ci.astype(jnp.int32)
    cjs = cjs.astype(jnp.int32)
    s2d = _scatter_call(cjs, ti, tjs)
    phi = _phi_call(ci, cjs, ti, tjs, W0, W1, W2)
    pHi = _mm_call(s2d, W0, W1, W2)
    return phi.reshape(B, 1), pHi
